# Initial kernel scaffold; baseline (speedup 1.0000x reference)
import jax
import jax.numpy as jnp
from jax.experimental import pallas as pl


def kernel(points, features, class_ids, params):
    raise NotImplementedError("write your pallas kernel here")


# speedup vs baseline: 1.1503x; 1.1503x over previous
import functools

import jax
import jax.numpy as jnp
from jax import lax
from jax.experimental import pallas as pl

NUM_CLASS_K = 16
B = 8
N0 = 2048


def _fold_cbr(p):
    s = p['g'] / jnp.sqrt(1.0 + 1e-5)
    return p['W'] * s[None, :], (p['b'] * s + p['be'])[None, :]


def _fps_body(npoint, n, x_ref, y_ref, z_ref, fi_ref, nx_ref, ny_ref, nz_ref):
    x = x_ref[...]
    y = y_ref[...]
    z = z_ref[...]
    iota_n = lax.broadcasted_iota(jnp.int32, (B, n), 1)
    iota_s = lax.broadcasted_iota(jnp.int32, (B, npoint), 1)
    fi_ref[...] = jnp.zeros((B, npoint), jnp.int32)
    nx_ref[...] = jnp.zeros((B, npoint), jnp.float32)
    ny_ref[...] = jnp.zeros((B, npoint), jnp.float32)
    nz_ref[...] = jnp.zeros((B, npoint), jnp.float32)

    def step(k, carry):
        dist, far = carry
        oh = iota_n == far
        cx = jnp.sum(jnp.where(oh, x, 0.0), axis=1, keepdims=True)
        cy = jnp.sum(jnp.where(oh, y, 0.0), axis=1, keepdims=True)
        cz = jnp.sum(jnp.where(oh, z, 0.0), axis=1, keepdims=True)
        sel = iota_s == k
        fi_ref[...] = jnp.where(sel, far, fi_ref[...])
        nx_ref[...] = jnp.where(sel, cx, nx_ref[...])
        ny_ref[...] = jnp.where(sel, cy, ny_ref[...])
        nz_ref[...] = jnp.where(sel, cz, nz_ref[...])
        d = (x - cx) ** 2 + (y - cy) ** 2 + (z - cz) ** 2
        dist = jnp.minimum(dist, d)
        m = jnp.max(dist, axis=1, keepdims=True)
        nf = jnp.min(jnp.where(dist == m, iota_n, n), axis=1, keepdims=True)
        return dist, nf

    lax.fori_loop(0, npoint, step,
                  (jnp.full((B, n), 1e10, jnp.float32),
                   jnp.zeros((B, 1), jnp.int32)))


def _fps(x, y, z, npoint):
    n = x.shape[1]
    return pl.pallas_call(
        functools.partial(_fps_body, npoint, n),
        out_shape=[jax.ShapeDtypeStruct((B, npoint), jnp.int32)] +
                  [jax.ShapeDtypeStruct((B, npoint), jnp.float32)] * 3,
    )(x, y, z)


def _mlp_max_body(ch, k, nlayer, *refs):
    g_ref = refs[0]
    w_refs = refs[1:1 + nlayer]
    b_refs = refs[1 + nlayer:1 + 2 * nlayer]
    o_ref = refs[1 + 2 * nlayer]
    x = g_ref[0]
    for i in range(nlayer):
        x = jnp.dot(x, w_refs[i][...], preferred_element_type=jnp.float32)
        x = jnp.maximum(x + b_refs[i][...], 0.0)
    c = x.shape[-1]
    o_ref[0] = jnp.max(x.reshape(ch, k, c), axis=1)


def _mlp_max(g, ws, bs, s, k):
    cin = g.shape[-1]
    cout = ws[-1].shape[1]
    ch = max(1, 4096 // k)
    nch = s // ch
    nl = len(ws)
    in_specs = [pl.BlockSpec((1, ch * k, cin), lambda b, c: (b, c, 0))]
    for w in ws:
        in_specs.append(pl.BlockSpec(w.shape, lambda b, c: (0, 0)))
    for bb in bs:
        in_specs.append(pl.BlockSpec(bb.shape, lambda b, c: (0, 0)))
    return pl.pallas_call(
        functools.partial(_mlp_max_body, ch, k, nl),
        grid=(B, nch),
        in_specs=in_specs,
        out_specs=pl.BlockSpec((1, ch, cout), lambda b, c: (b, c, 0)),
        out_shape=jax.ShapeDtypeStruct((B, s, cout), jnp.float32),
    )(g, *ws, *bs)


def _gsa_body(nlayer, *refs):
    x_ref = refs[0]
    w_refs = refs[1:1 + nlayer]
    b_refs = refs[1 + nlayer:1 + 2 * nlayer]
    o_ref = refs[1 + 2 * nlayer]
    x = x_ref[0]
    for i in range(nlayer):
        x = jnp.dot(x, w_refs[i][...], preferred_element_type=jnp.float32)
        x = jnp.maximum(x + b_refs[i][...], 0.0)
    o_ref[0] = jnp.max(x, axis=0, keepdims=True)


def _gsa(x, ws, bs):
    s, cin = x.shape[1], x.shape[2]
    cout = ws[-1].shape[1]
    nl = len(ws)
    in_specs = [pl.BlockSpec((1, s, cin), lambda b: (b, 0, 0))]
    in_specs += [pl.BlockSpec(w.shape, lambda b: (0, 0)) for w in ws]
    in_specs += [pl.BlockSpec(bb.shape, lambda b: (0, 0)) for bb in bs]
    return pl.pallas_call(
        functools.partial(_gsa_body, nl),
        grid=(B,),
        in_specs=in_specs,
        out_specs=pl.BlockSpec((1, 1, cout), lambda b: (b, 0, 0)),
        out_shape=jax.ShapeDtypeStruct((B, 1, cout), jnp.float32),
    )(x, *ws, *bs)


def _fp1_body(f_ref, g_ref, w1a_ref, w1b_ref, b1_ref, w2_ref, b2_ref, o_ref):
    f = f_ref[0]
    gtop = jnp.dot(g_ref[0], w1b_ref[...], preferred_element_type=jnp.float32)
    h = jnp.dot(f, w1a_ref[...], preferred_element_type=jnp.float32)
    h = jnp.maximum(h + gtop + b1_ref[...], 0.0)
    h = jnp.dot(h, w2_ref[...], preferred_element_type=jnp.float32)
    o_ref[0] = jnp.maximum(h + b2_ref[...], 0.0)


def _fp1(feats1, gvec, w1, b1, w2, b2):
    s, c1 = feats1.shape[1], feats1.shape[2]
    cg = gvec.shape[-1]
    w1a, w1b = w1[:c1], w1[c1:]
    cout = w2.shape[1]
    return pl.pallas_call(
        _fp1_body,
        grid=(B,),
        in_specs=[
            pl.BlockSpec((1, s, c1), lambda b: (b, 0, 0)),
            pl.BlockSpec((1, 1, cg), lambda b: (b, 0, 0)),
            pl.BlockSpec(w1a.shape, lambda b: (0, 0)),
            pl.BlockSpec(w1b.shape, lambda b: (0, 0)),
            pl.BlockSpec(b1.shape, lambda b: (0, 0)),
            pl.BlockSpec(w2.shape, lambda b: (0, 0)),
            pl.BlockSpec(b2.shape, lambda b: (0, 0)),
        ],
        out_specs=pl.BlockSpec((1, s, cout), lambda b: (b, 0, 0)),
        out_shape=jax.ShapeDtypeStruct((B, s, cout), jnp.float32),
    )(feats1, gvec, w1a, w1b, b1, w2, b2)


def _interp_w(p1, p2t):
    d = ((p1[:, 0:1] - p2t[0:1, :]) ** 2
         + (p1[:, 1:2] - p2t[1:2, :]) ** 2
         + (p1[:, 2:3] - p2t[2:3, :]) ** 2)
    big = jnp.float32(3e38)
    t = d
    m1 = jnp.min(t, axis=1, keepdims=True)
    t = jnp.where(t == m1, big, t)
    m2 = jnp.min(t, axis=1, keepdims=True)
    t = jnp.where(t == m2, big, t)
    m3 = jnp.min(t, axis=1, keepdims=True)
    mask = d <= m3
    recip = jnp.where(mask, 1.0 / (d + 1e-8), 0.0)
    return recip / jnp.sum(recip, axis=1, keepdims=True)


def _fp2_body(p1_ref, p2t_ref, f1_ref, f2_ref,
              w1a_ref, w1b_ref, b1_ref, w2_ref, b2_ref, o_ref):
    w = _interp_w(p1_ref[0], p2t_ref[0])
    interp = jnp.dot(w, f2_ref[0], preferred_element_type=jnp.float32)
    h = (jnp.dot(f1_ref[0], w1a_ref[...], preferred_element_type=jnp.float32)
         + jnp.dot(interp, w1b_ref[...], preferred_element_type=jnp.float32))
    h = jnp.maximum(h + b1_ref[...], 0.0)
    h = jnp.dot(h, w2_ref[...], preferred_element_type=jnp.float32)
    o_ref[0] = jnp.maximum(h + b2_ref[...], 0.0)


def _fp2(p1, p2t, feats1, feats2, w1, b1, w2, b2):
    s1, s2 = p1.shape[1], p2t.shape[2]
    c1, c2 = feats1.shape[2], feats2.shape[2]
    w1a, w1b = w1[:c1], w1[c1:]
    cout = w2.shape[1]
    return pl.pallas_call(
        _fp2_body,
        grid=(B,),
        in_specs=[
            pl.BlockSpec((1, s1, 3), lambda b: (b, 0, 0)),
            pl.BlockSpec((1, 3, s2), lambda b: (b, 0, 0)),
            pl.BlockSpec((1, s1, c1), lambda b: (b, 0, 0)),
            pl.BlockSpec((1, s2, c2), lambda b: (b, 0, 0)),
            pl.BlockSpec(w1a.shape, lambda b: (0, 0)),
            pl.BlockSpec(w1b.shape, lambda b: (0, 0)),
            pl.BlockSpec(b1.shape, lambda b: (0, 0)),
            pl.BlockSpec(w2.shape, lambda b: (0, 0)),
            pl.BlockSpec(b2.shape, lambda b: (0, 0)),
        ],
        out_specs=pl.BlockSpec((1, s1, cout), lambda b: (b, 0, 0)),
        out_shape=jax.ShapeDtypeStruct((B, s1, cout), jnp.float32),
    )(p1, p2t, feats1, feats2, w1a, w1b, b1, w2, b2)


def _fp3_head(p1, p2t, oh, f0t, feats2, w1, b1, w2, b2, wc1, bc1, wc2, bc2):
    s1, s2 = p1.shape[1], p2t.shape[2]
    c2 = feats2.shape[2]
    woh, wf, wx, wi = (w1[:NUM_CLASS_K], w1[NUM_CLASS_K:NUM_CLASS_K + 3],
                       w1[NUM_CLASS_K + 3:NUM_CLASS_K + 6],
                       w1[NUM_CLASS_K + 6:])
    npart = wc2.shape[1]

    def body(p1_ref, p2t_ref, oh_ref, f0_ref, f2_ref,
             woh_ref, wf_ref, wx_ref, wi_ref, b1_ref, w2_ref, b2_ref,
             wc1_ref, bc1_ref, wc2_ref, bc2_ref, o_ref):
        w = _interp_w(p1_ref[0], p2t_ref[0])
        interp = jnp.dot(w, f2_ref[0], preferred_element_type=jnp.float32)
        ohrow = jnp.dot(oh_ref[0], woh_ref[...],
                        preferred_element_type=jnp.float32)
        h = (jnp.dot(f0_ref[0], wf_ref[...], preferred_element_type=jnp.float32)
             + jnp.dot(p1_ref[0], wx_ref[...], preferred_element_type=jnp.float32)
             + jnp.dot(interp, wi_ref[...], preferred_element_type=jnp.float32))
        h = jnp.maximum(h + ohrow + b1_ref[...], 0.0)
        h = jnp.maximum(jnp.dot(h, w2_ref[...],
                                preferred_element_type=jnp.float32)
                        + b2_ref[...], 0.0)
        h = jnp.maximum(jnp.dot(h, wc1_ref[...],
                                preferred_element_type=jnp.float32)
                        + bc1_ref[...], 0.0)
        out = lax.dot_general(wc2_ref[...], h, (((0,), (1,)), ((), ())),
                              preferred_element_type=jnp.float32)
        o_ref[0] = out + bc2_ref[...].reshape(npart, 1)

    return pl.pallas_call(
        body,
        grid=(B,),
        in_specs=[
            pl.BlockSpec((1, s1, 3), lambda b: (b, 0, 0)),
            pl.BlockSpec((1, 3, s2), lambda b: (b, 0, 0)),
            pl.BlockSpec((1, 1, NUM_CLASS_K), lambda b: (b, 0, 0)),
            pl.BlockSpec((1, s1, 3), lambda b: (b, 0, 0)),
            pl.BlockSpec((1, s2, c2), lambda b: (b, 0, 0)),
            pl.BlockSpec(woh.shape, lambda b: (0, 0)),
            pl.BlockSpec(wf.shape, lambda b: (0, 0)),
            pl.BlockSpec(wx.shape, lambda b: (0, 0)),
            pl.BlockSpec(wi.shape, lambda b: (0, 0)),
            pl.BlockSpec(b1.shape, lambda b: (0, 0)),
            pl.BlockSpec(w2.shape, lambda b: (0, 0)),
            pl.BlockSpec(b2.shape, lambda b: (0, 0)),
            pl.BlockSpec(wc1.shape, lambda b: (0, 0)),
            pl.BlockSpec(bc1.shape, lambda b: (0, 0)),
            pl.BlockSpec(wc2.shape, lambda b: (0, 0)),
            pl.BlockSpec(bc2.shape, lambda b: (0, 0)),
        ],
        out_specs=pl.BlockSpec((1, npart, s1), lambda b: (b, 0, 0)),
        out_shape=jax.ShapeDtypeStruct((B, npart, s1), jnp.float32),
    )(p1, p2t, oh, f0t, feats2, woh, wf, wx, wi, b1, w2, b2,
      wc1, bc1, wc2, bc2)


def _ball_glue(r, k, xyz, new_xyz):
    n = xyz.shape[1]
    sqr = jnp.sum((new_xyz[:, :, None, :] - xyz[:, None, :, :]) ** 2, axis=-1)
    gid = jnp.where(sqr > r * r, n,
                    jnp.arange(n, dtype=jnp.int32)[None, None, :])
    gid = jnp.sort(gid, axis=-1)[:, :, :k]
    first = gid[:, :, 0:1]
    return jnp.where(gid == n, first, gid)


def _gather_pts(pts, idx):
    bidx = jnp.arange(pts.shape[0]).reshape((-1,) + (1,) * (idx.ndim - 1))
    return pts[bidx, idx]


def kernel(points, features, class_ids, params):
    x, y, z = points[:, 0], points[:, 1], points[:, 2]
    f0t = jnp.transpose(features, (0, 2, 1))
    xyz = jnp.stack([x, y, z], axis=-1)
    oh = jax.nn.one_hot(class_ids, NUM_CLASS_K, dtype=jnp.float32)[:, None, :]

    ms1 = [[_fold_cbr(p) for p in mlp] for mlp in params['ms1']]
    ms2 = [[_fold_cbr(p) for p in mlp] for mlp in params['ms2']]
    gsa_p = [_fold_cbr(p) for p in params['gsa']]
    fp1_p = [_fold_cbr(p) for p in params['fp1']]
    fp2_p = [_fold_cbr(p) for p in params['fp2']]
    fp3_p = [_fold_cbr(p) for p in params['fp3']]
    wc1, bc1 = _fold_cbr(params['cls1'])
    wc2 = params['cls2']['W']
    bc2 = params['cls2']['b'][None, :]

    _, nx, ny, nz = _fps(x, y, z, 512)
    l1_xyz = jnp.stack([nx, ny, nz], axis=-1)
    outs = []
    for r, k, mlp in zip([0.1, 0.2, 0.4], [32, 64, 128], ms1):
        idx = _ball_glue(r, k, xyz, l1_xyz)
        gx = _gather_pts(xyz, idx) - l1_xyz[:, :, None, :]
        gf = _gather_pts(f0t, idx)
        g = jnp.concatenate([gf, gx], axis=-1).reshape(B, 512 * k, 6)
        ws = [w for w, _ in mlp]
        bs = [b for _, b in mlp]
        outs.append(_mlp_max(g, ws, bs, 512, k))
    l1_f = jnp.concatenate(outs, axis=-1)

    _, nx2, ny2, nz2 = _fps(nx, ny, nz, 128)
    l2_xyz = jnp.stack([nx2, ny2, nz2], axis=-1)
    l1_cat = jnp.concatenate([l1_f, l1_xyz], axis=-1)
    outs2 = []
    for r, k, mlp in zip([0.4, 0.8], [64, 128], ms2):
        idx = _ball_glue(r, k, l1_xyz, l2_xyz)
        g = _gather_pts(l1_cat, idx)
        g = g.at[:, :, :, 320:].add(-l2_xyz[:, :, None, :])
        g = g.reshape(B, 128 * k, 323)
        ws = [w for w, _ in mlp]
        bs = [b for _, b in mlp]
        outs2.append(_mlp_max(g, ws, bs, 128, k))
    l2_f = jnp.concatenate(outs2, axis=-1)

    x2 = jnp.concatenate([l2_f, l2_xyz], axis=-1)
    gvec = _gsa(x2, [w for w, _ in gsa_p], [b for _, b in gsa_p])

    l2_xyzt = jnp.stack([nx2, ny2, nz2], axis=1)
    l1_xyzt = jnp.stack([nx, ny, nz], axis=1)
    l2_fn = _fp1(l2_f, gvec, fp1_p[0][0], fp1_p[0][1],
                 fp1_p[1][0], fp1_p[1][1])
    l1_fn = _fp2(l1_xyz, l2_xyzt, l1_f, l2_fn,
                 fp2_p[0][0], fp2_p[0][1], fp2_p[1][0], fp2_p[1][1])
    out = _fp3_head(xyz, l1_xyzt, oh, f0t, l1_fn,
                    fp3_p[0][0], fp3_p[0][1], fp3_p[1][0], fp3_p[1][1],
                    wc1, bc1, wc2, bc2)
    return out


# speedup vs baseline: 1.1533x; 1.0026x over previous
import functools

import jax
import jax.numpy as jnp
from jax import lax
from jax.experimental import pallas as pl
from jax.experimental.pallas import tpu as pltpu
from jax.experimental.pallas import tpu_sc as plsc

NUM_CLASS_K = 16
B = 8
N0 = 2048


def _fold_cbr(p):
    s = p['g'] / jnp.sqrt(1.0 + 1e-5)
    return p['W'] * s[None, :], (p['b'] * s + p['be'])[None, :]


def _fps_body(npoint, n, x_ref, y_ref, z_ref, fi_ref, nx_ref, ny_ref, nz_ref):
    x = x_ref[...]
    y = y_ref[...]
    z = z_ref[...]
    iota_n = lax.broadcasted_iota(jnp.int32, (B, n), 1)
    iota_s = lax.broadcasted_iota(jnp.int32, (B, npoint), 1)
    fi_ref[...] = jnp.zeros((B, npoint), jnp.int32)
    nx_ref[...] = jnp.zeros((B, npoint), jnp.float32)
    ny_ref[...] = jnp.zeros((B, npoint), jnp.float32)
    nz_ref[...] = jnp.zeros((B, npoint), jnp.float32)

    def step(k, carry):
        dist, far = carry
        oh = iota_n == far
        cx = jnp.sum(jnp.where(oh, x, 0.0), axis=1, keepdims=True)
        cy = jnp.sum(jnp.where(oh, y, 0.0), axis=1, keepdims=True)
        cz = jnp.sum(jnp.where(oh, z, 0.0), axis=1, keepdims=True)
        sel = iota_s == k
        fi_ref[...] = jnp.where(sel, far, fi_ref[...])
        nx_ref[...] = jnp.where(sel, cx, nx_ref[...])
        ny_ref[...] = jnp.where(sel, cy, ny_ref[...])
        nz_ref[...] = jnp.where(sel, cz, nz_ref[...])
        d = (x - cx) ** 2 + (y - cy) ** 2 + (z - cz) ** 2
        dist = jnp.minimum(dist, d)
        m = jnp.max(dist, axis=1, keepdims=True)
        nf = jnp.min(jnp.where(dist == m, iota_n, n), axis=1, keepdims=True)
        return dist, nf

    lax.fori_loop(0, npoint, step,
                  (jnp.full((B, n), 1e10, jnp.float32),
                   jnp.zeros((B, 1), jnp.int32)))


def _fps(x, y, z, npoint):
    n = x.shape[1]
    return pl.pallas_call(
        functools.partial(_fps_body, npoint, n),
        out_shape=[jax.ShapeDtypeStruct((B, npoint), jnp.int32)] +
                  [jax.ShapeDtypeStruct((B, npoint), jnp.float32)] * 3,
    )(x, y, z)


def _mlp_max_body(ch, k, nlayer, *refs):
    g_ref = refs[0]
    w_refs = refs[1:1 + nlayer]
    b_refs = refs[1 + nlayer:1 + 2 * nlayer]
    o_ref = refs[1 + 2 * nlayer]
    x = g_ref[0]
    for i in range(nlayer):
        x = jnp.dot(x, w_refs[i][...], preferred_element_type=jnp.float32)
        x = jnp.maximum(x + b_refs[i][...], 0.0)
    c = x.shape[-1]
    o_ref[0] = jnp.max(x.reshape(ch, k, c), axis=1)


def _mlp_max(g, ws, bs, s, k):
    cin = g.shape[-1]
    cout = ws[-1].shape[1]
    ch = max(1, 4096 // k)
    nch = s // ch
    nl = len(ws)
    in_specs = [pl.BlockSpec((1, ch * k, cin), lambda b, c: (b, c, 0))]
    for w in ws:
        in_specs.append(pl.BlockSpec(w.shape, lambda b, c: (0, 0)))
    for bb in bs:
        in_specs.append(pl.BlockSpec(bb.shape, lambda b, c: (0, 0)))
    return pl.pallas_call(
        functools.partial(_mlp_max_body, ch, k, nl),
        grid=(B, nch),
        in_specs=in_specs,
        out_specs=pl.BlockSpec((1, ch, cout), lambda b, c: (b, c, 0)),
        out_shape=jax.ShapeDtypeStruct((B, s, cout), jnp.float32),
    )(g, *ws, *bs)


def _gsa_body(nlayer, *refs):
    x_ref = refs[0]
    w_refs = refs[1:1 + nlayer]
    b_refs = refs[1 + nlayer:1 + 2 * nlayer]
    o_ref = refs[1 + 2 * nlayer]
    x = x_ref[0]
    for i in range(nlayer):
        x = jnp.dot(x, w_refs[i][...], preferred_element_type=jnp.float32)
        x = jnp.maximum(x + b_refs[i][...], 0.0)
    o_ref[0] = jnp.max(x, axis=0, keepdims=True)


def _gsa(x, ws, bs):
    s, cin = x.shape[1], x.shape[2]
    cout = ws[-1].shape[1]
    nl = len(ws)
    in_specs = [pl.BlockSpec((1, s, cin), lambda b: (b, 0, 0))]
    in_specs += [pl.BlockSpec(w.shape, lambda b: (0, 0)) for w in ws]
    in_specs += [pl.BlockSpec(bb.shape, lambda b: (0, 0)) for bb in bs]
    return pl.pallas_call(
        functools.partial(_gsa_body, nl),
        grid=(B,),
        in_specs=in_specs,
        out_specs=pl.BlockSpec((1, 1, cout), lambda b: (b, 0, 0)),
        out_shape=jax.ShapeDtypeStruct((B, 1, cout), jnp.float32),
    )(x, *ws, *bs)


def _fp1_body(f_ref, g_ref, w1a_ref, w1b_ref, b1_ref, w2_ref, b2_ref, o_ref):
    f = f_ref[0]
    gtop = jnp.dot(g_ref[0], w1b_ref[...], preferred_element_type=jnp.float32)
    h = jnp.dot(f, w1a_ref[...], preferred_element_type=jnp.float32)
    h = jnp.maximum(h + gtop + b1_ref[...], 0.0)
    h = jnp.dot(h, w2_ref[...], preferred_element_type=jnp.float32)
    o_ref[0] = jnp.maximum(h + b2_ref[...], 0.0)


def _fp1(feats1, gvec, w1, b1, w2, b2):
    s, c1 = feats1.shape[1], feats1.shape[2]
    cg = gvec.shape[-1]
    w1a, w1b = w1[:c1], w1[c1:]
    cout = w2.shape[1]
    return pl.pallas_call(
        _fp1_body,
        grid=(B,),
        in_specs=[
            pl.BlockSpec((1, s, c1), lambda b: (b, 0, 0)),
            pl.BlockSpec((1, 1, cg), lambda b: (b, 0, 0)),
            pl.BlockSpec(w1a.shape, lambda b: (0, 0)),
            pl.BlockSpec(w1b.shape, lambda b: (0, 0)),
            pl.BlockSpec(b1.shape, lambda b: (0, 0)),
            pl.BlockSpec(w2.shape, lambda b: (0, 0)),
            pl.BlockSpec(b2.shape, lambda b: (0, 0)),
        ],
        out_specs=pl.BlockSpec((1, s, cout), lambda b: (b, 0, 0)),
        out_shape=jax.ShapeDtypeStruct((B, s, cout), jnp.float32),
    )(feats1, gvec, w1a, w1b, b1, w2, b2)


def _interp_w(p1, p2t):
    d = ((p1[:, 0:1] - p2t[0:1, :]) ** 2
         + (p1[:, 1:2] - p2t[1:2, :]) ** 2
         + (p1[:, 2:3] - p2t[2:3, :]) ** 2)
    big = jnp.float32(3e38)
    t = d
    m1 = jnp.min(t, axis=1, keepdims=True)
    t = jnp.where(t == m1, big, t)
    m2 = jnp.min(t, axis=1, keepdims=True)
    t = jnp.where(t == m2, big, t)
    m3 = jnp.min(t, axis=1, keepdims=True)
    mask = d <= m3
    recip = jnp.where(mask, 1.0 / (d + 1e-8), 0.0)
    return recip / jnp.sum(recip, axis=1, keepdims=True)


def _fp2_body(p1_ref, p2t_ref, f1_ref, f2_ref,
              w1a_ref, w1b_ref, b1_ref, w2_ref, b2_ref, o_ref):
    w = _interp_w(p1_ref[0], p2t_ref[0])
    interp = jnp.dot(w, f2_ref[0], preferred_element_type=jnp.float32)
    h = (jnp.dot(f1_ref[0], w1a_ref[...], preferred_element_type=jnp.float32)
         + jnp.dot(interp, w1b_ref[...], preferred_element_type=jnp.float32))
    h = jnp.maximum(h + b1_ref[...], 0.0)
    h = jnp.dot(h, w2_ref[...], preferred_element_type=jnp.float32)
    o_ref[0] = jnp.maximum(h + b2_ref[...], 0.0)


def _fp2(p1, p2t, feats1, feats2, w1, b1, w2, b2):
    s1, s2 = p1.shape[1], p2t.shape[2]
    c1, c2 = feats1.shape[2], feats2.shape[2]
    w1a, w1b = w1[:c1], w1[c1:]
    cout = w2.shape[1]
    return pl.pallas_call(
        _fp2_body,
        grid=(B,),
        in_specs=[
            pl.BlockSpec((1, s1, 3), lambda b: (b, 0, 0)),
            pl.BlockSpec((1, 3, s2), lambda b: (b, 0, 0)),
            pl.BlockSpec((1, s1, c1), lambda b: (b, 0, 0)),
            pl.BlockSpec((1, s2, c2), lambda b: (b, 0, 0)),
            pl.BlockSpec(w1a.shape, lambda b: (0, 0)),
            pl.BlockSpec(w1b.shape, lambda b: (0, 0)),
            pl.BlockSpec(b1.shape, lambda b: (0, 0)),
            pl.BlockSpec(w2.shape, lambda b: (0, 0)),
            pl.BlockSpec(b2.shape, lambda b: (0, 0)),
        ],
        out_specs=pl.BlockSpec((1, s1, cout), lambda b: (b, 0, 0)),
        out_shape=jax.ShapeDtypeStruct((B, s1, cout), jnp.float32),
    )(p1, p2t, feats1, feats2, w1a, w1b, b1, w2, b2)


def _fp3_head(p1, p2t, oh, f0t, feats2, w1, b1, w2, b2, wc1, bc1, wc2, bc2):
    s1, s2 = p1.shape[1], p2t.shape[2]
    c2 = feats2.shape[2]
    woh, wf, wx, wi = (w1[:NUM_CLASS_K], w1[NUM_CLASS_K:NUM_CLASS_K + 3],
                       w1[NUM_CLASS_K + 3:NUM_CLASS_K + 6],
                       w1[NUM_CLASS_K + 6:])
    npart = wc2.shape[1]

    def body(p1_ref, p2t_ref, oh_ref, f0_ref, f2_ref,
             woh_ref, wf_ref, wx_ref, wi_ref, b1_ref, w2_ref, b2_ref,
             wc1_ref, bc1_ref, wc2_ref, bc2_ref, o_ref):
        w = _interp_w(p1_ref[0], p2t_ref[0])
        interp = jnp.dot(w, f2_ref[0], preferred_element_type=jnp.float32)
        ohrow = jnp.dot(oh_ref[0], woh_ref[...],
                        preferred_element_type=jnp.float32)
        h = (jnp.dot(f0_ref[0], wf_ref[...], preferred_element_type=jnp.float32)
             + jnp.dot(p1_ref[0], wx_ref[...], preferred_element_type=jnp.float32)
             + jnp.dot(interp, wi_ref[...], preferred_element_type=jnp.float32))
        h = jnp.maximum(h + ohrow + b1_ref[...], 0.0)
        h = jnp.maximum(jnp.dot(h, w2_ref[...],
                                preferred_element_type=jnp.float32)
                        + b2_ref[...], 0.0)
        h = jnp.maximum(jnp.dot(h, wc1_ref[...],
                                preferred_element_type=jnp.float32)
                        + bc1_ref[...], 0.0)
        out = lax.dot_general(wc2_ref[...], h, (((0,), (1,)), ((), ())),
                              preferred_element_type=jnp.float32)
        o_ref[0] = out + bc2_ref[...].reshape(npart, 1)

    return pl.pallas_call(
        body,
        grid=(B,),
        in_specs=[
            pl.BlockSpec((1, s1, 3), lambda b: (b, 0, 0)),
            pl.BlockSpec((1, 3, s2), lambda b: (b, 0, 0)),
            pl.BlockSpec((1, 1, NUM_CLASS_K), lambda b: (b, 0, 0)),
            pl.BlockSpec((1, s1, 3), lambda b: (b, 0, 0)),
            pl.BlockSpec((1, s2, c2), lambda b: (b, 0, 0)),
            pl.BlockSpec(woh.shape, lambda b: (0, 0)),
            pl.BlockSpec(wf.shape, lambda b: (0, 0)),
            pl.BlockSpec(wx.shape, lambda b: (0, 0)),
            pl.BlockSpec(wi.shape, lambda b: (0, 0)),
            pl.BlockSpec(b1.shape, lambda b: (0, 0)),
            pl.BlockSpec(w2.shape, lambda b: (0, 0)),
            pl.BlockSpec(b2.shape, lambda b: (0, 0)),
            pl.BlockSpec(wc1.shape, lambda b: (0, 0)),
            pl.BlockSpec(bc1.shape, lambda b: (0, 0)),
            pl.BlockSpec(wc2.shape, lambda b: (0, 0)),
            pl.BlockSpec(bc2.shape, lambda b: (0, 0)),
        ],
        out_specs=pl.BlockSpec((1, npart, s1), lambda b: (b, 0, 0)),
        out_shape=jax.ShapeDtypeStruct((B, npart, s1), jnp.float32),
    )(p1, p2t, oh, f0t, feats2, woh, wf, wx, wi, b1, w2, b2,
      wc1, bc1, wc2, bc2)


def _ballq_sc(px, py, pz, cx, cy, cz, radii, ks):
    n = px.shape[1]
    s = cx.shape[1]
    nw = 32
    per = (B * s) // nw
    nch = n // 16
    nbr = len(radii)
    r2 = [jnp.float32(r * r) for r in radii]
    mesh = plsc.VectorSubcoreMesh(core_axis_name="c", subcore_axis_name="s")

    def body(px_h, py_h, pz_h, cx_h, cy_h, cz_h, *rest):
        iota = lax.iota(jnp.int32, 16)
        out_hs = rest[:nbr]
        xs, ys, zs, cxs, cys, czs = rest[nbr:nbr + 6]
        bufs = rest[nbr + 6:nbr + 6 + nbr]
        cbufs = rest[nbr + 6 + nbr:nbr + 6 + 2 * nbr]
        shs = rest[nbr + 6 + 2 * nbr]
        w = lax.axis_index("s") * 2 + lax.axis_index("c")
        b = (w * per) // s
        lo = (w * per) % s
        pltpu.sync_copy(px_h.at[b], xs)
        pltpu.sync_copy(py_h.at[b], ys)
        pltpu.sync_copy(pz_h.at[b], zs)
        pltpu.sync_copy(cx_h.at[b, pl.ds(lo, per)], cxs.at[pl.ds(0, per)])
        pltpu.sync_copy(cy_h.at[b, pl.ds(lo, per)], cys.at[pl.ds(0, per)])
        pltpu.sync_copy(cz_h.at[b, pl.ds(lo, per)], czs.at[pl.ds(0, per)])

        for a in range(3):
            shs[pl.ds(a * 48, 16)] = jnp.zeros((16,), jnp.int32)
            shs[pl.ds(a * 48 + 32, 16)] = jnp.zeros((16,), jnp.int32)

        def psum16(r, a):
            for t in (1, 2, 4, 8):
                shs[pl.ds(a * 48 + 16, 16)] = r
                r = r + shs[pl.ds(a * 48 + 16 - t, 16)]
            return r

        def compact16(v, disp):
            for t in (1, 2, 4, 8):
                shs[pl.ds(48 + 16, 16)] = v
                shs[pl.ds(96 + 16, 16)] = disp
                vsh = shs[pl.ds(48 + 16 + t, 16)]
                dsh = shs[pl.ds(96 + 16 + t, 16)]
                cond = (dsh & t) != 0
                v = jnp.where(cond, vsh, v)
                disp = jnp.where(cond, dsh - t, disp)
            return v

        def center_step(i, _):
            cxi = cxs[pl.ds(i, 16)][0]
            cyi = cys[pl.ds(i, 16)][0]
            czi = czs[pl.ds(i, 16)][0]

            def chunk_step(c, os_):
                xv = xs[pl.ds(c * 16, 16)]
                yv = ys[pl.ds(c * 16, 16)]
                zv = zs[pl.ds(c * 16, 16)]
                dx = xv - cxi
                dy = yv - cyi
                dz = zv - czi
                d = dx * dx + dy * dy + dz * dz
                idxv = c * 16 + iota
                new_os = []
                for j in range(nbr):
                    k = ks[j]
                    oj = os_[j]
                    m = d <= r2[j]

                    def do(o, m=m, j=j, k=k, idxv=idxv):
                        mi = jnp.where(m, 1, 0)
                        r = psum16(mi, 0)
                        cnt = r[15]

                        def wr(o, m=m, j=j, k=k, idxv=idxv, r=r, cnt=cnt):
                            disp = jnp.where(m, iota - (r - 1), 0)
                            v = compact16(idxv, disp)
                            bufs[j][pl.ds(i * (k + 16) + o, 16)] = v
                            return jnp.minimum(o + cnt, k)

                        return lax.cond(cnt > 0, wr, lambda o: o, o)

                    new_os.append(lax.cond(oj < k, do, lambda o: o, oj))
                return tuple(new_os)

            os_ = lax.fori_loop(0, nch, chunk_step,
                                tuple(jnp.int32(0) for _ in range(nbr)))
            for j in range(nbr):
                k = ks[j]
                first = bufs[j][pl.ds(i * (k + 16), 16)][0]
                for kc in range(k // 16):
                    pos = kc * 16 + iota
                    cur = bufs[j][pl.ds(i * (k + 16) + kc * 16, 16)]
                    cbufs[j][pl.ds(i * k + kc * 16, 16)] = jnp.where(
                        pos >= os_[j], first, cur)
            return 0

        lax.fori_loop(0, per, center_step, 0)
        for j in range(nbr):
            k = ks[j]
            pltpu.sync_copy(cbufs[j],
                            out_hs[j].at[pl.ds((b * s + lo) * k, per * k)])

    fn = pl.kernel(
        body,
        out_type=[jax.ShapeDtypeStruct((B * s * k,), jnp.int32) for k in ks],
        mesh=mesh,
        scratch_types=(
            [pltpu.VMEM((n,), jnp.float32)] * 3
            + [pltpu.VMEM((per + 16,), jnp.float32)] * 3
            + [pltpu.VMEM((per * (k + 16),), jnp.int32) for k in ks]
            + [pltpu.VMEM((per * k,), jnp.int32) for k in ks]
            + [pltpu.VMEM((144,), jnp.int32)]),
    )
    outs = fn(px, py, pz, cx, cy, cz)
    return [o.reshape(B, s, k) for o, k in zip(outs, ks)]


def _ball_glue(r, k, xyz, new_xyz):
    n = xyz.shape[1]
    sqr = jnp.sum((new_xyz[:, :, None, :] - xyz[:, None, :, :]) ** 2, axis=-1)
    gid = jnp.where(sqr > r * r, n,
                    jnp.arange(n, dtype=jnp.int32)[None, None, :])
    gid = jnp.sort(gid, axis=-1)[:, :, :k]
    first = gid[:, :, 0:1]
    return jnp.where(gid == n, first, gid)


def _gather_pts(pts, idx):
    bidx = jnp.arange(pts.shape[0]).reshape((-1,) + (1,) * (idx.ndim - 1))
    return pts[bidx, idx]


def kernel(points, features, class_ids, params):
    x, y, z = points[:, 0], points[:, 1], points[:, 2]
    f0t = jnp.transpose(features, (0, 2, 1))
    xyz = jnp.stack([x, y, z], axis=-1)
    oh = jax.nn.one_hot(class_ids, NUM_CLASS_K, dtype=jnp.float32)[:, None, :]

    ms1 = [[_fold_cbr(p) for p in mlp] for mlp in params['ms1']]
    ms2 = [[_fold_cbr(p) for p in mlp] for mlp in params['ms2']]
    gsa_p = [_fold_cbr(p) for p in params['gsa']]
    fp1_p = [_fold_cbr(p) for p in params['fp1']]
    fp2_p = [_fold_cbr(p) for p in params['fp2']]
    fp3_p = [_fold_cbr(p) for p in params['fp3']]
    wc1, bc1 = _fold_cbr(params['cls1'])
    wc2 = params['cls2']['W']
    bc2 = params['cls2']['b'][None, :]

    _, nx, ny, nz = _fps(x, y, z, 512)
    l1_xyz = jnp.stack([nx, ny, nz], axis=-1)
    idxs1 = _ballq_sc(x, y, z, nx, ny, nz, [0.1, 0.2, 0.4], [32, 64, 128])
    outs = []
    for idx, k, mlp in zip(idxs1, [32, 64, 128], ms1):
        gx = _gather_pts(xyz, idx) - l1_xyz[:, :, None, :]
        gf = _gather_pts(f0t, idx)
        g = jnp.concatenate([gf, gx], axis=-1).reshape(B, 512 * k, 6)
        ws = [w for w, _ in mlp]
        bs = [b for _, b in mlp]
        outs.append(_mlp_max(g, ws, bs, 512, k))
    l1_f = jnp.concatenate(outs, axis=-1)

    _, nx2, ny2, nz2 = _fps(nx, ny, nz, 128)
    l2_xyz = jnp.stack([nx2, ny2, nz2], axis=-1)
    l1_cat = jnp.concatenate([l1_f, l1_xyz], axis=-1)
    idxs2 = _ballq_sc(nx, ny, nz, nx2, ny2, nz2, [0.4, 0.8], [64, 128])
    outs2 = []
    for idx, k, mlp in zip(idxs2, [64, 128], ms2):
        g = _gather_pts(l1_cat, idx)
        g = g.at[:, :, :, 320:].add(-l2_xyz[:, :, None, :])
        g = g.reshape(B, 128 * k, 323)
        ws = [w for w, _ in mlp]
        bs = [b for _, b in mlp]
        outs2.append(_mlp_max(g, ws, bs, 128, k))
    l2_f = jnp.concatenate(outs2, axis=-1)

    x2 = jnp.concatenate([l2_f, l2_xyz], axis=-1)
    gvec = _gsa(x2, [w for w, _ in gsa_p], [b for _, b in gsa_p])

    l2_xyzt = jnp.stack([nx2, ny2, nz2], axis=1)
    l1_xyzt = jnp.stack([nx, ny, nz], axis=1)
    l2_fn = _fp1(l2_f, gvec, fp1_p[0][0], fp1_p[0][1],
                 fp1_p[1][0], fp1_p[1][1])
    l1_fn = _fp2(l1_xyz, l2_xyzt, l1_f, l2_fn,
                 fp2_p[0][0], fp2_p[0][1], fp2_p[1][0], fp2_p[1][1])
    out = _fp3_head(xyz, l1_xyzt, oh, f0t, l1_fn,
                    fp3_p[0][0], fp3_p[0][1], fp3_p[1][0], fp3_p[1][1],
                    wc1, bc1, wc2, bc2)
    return out


# speedup vs baseline: 12.1006x; 10.4925x over previous
import functools

import jax
import jax.numpy as jnp
from jax import lax
from jax.experimental import pallas as pl
from jax.experimental.pallas import tpu as pltpu
from jax.experimental.pallas import tpu_sc as plsc

NUM_CLASS_K = 16
B = 8
N0 = 2048


def _fold_cbr(p):
    s = p['g'] / jnp.sqrt(1.0 + 1e-5)
    return p['W'] * s[None, :], (p['b'] * s + p['be'])[None, :]


def _padc(a, n):
    return jnp.concatenate(
        [a, jnp.zeros(a.shape[:-1] + (n - a.shape[-1],), jnp.float32)],
        axis=-1)


def _padrc(a, nr, nc):
    a = jnp.concatenate(
        [a, jnp.zeros((nr - a.shape[0], a.shape[1]), jnp.float32)], axis=0)
    return _padc(a, nc)


def _fps_body(npoint, n, x_ref, y_ref, z_ref, fi_ref, nx_ref, ny_ref, nz_ref):
    x = x_ref[...]
    y = y_ref[...]
    z = z_ref[...]
    iota_n = lax.broadcasted_iota(jnp.int32, (B, n), 1)
    iota_s = lax.broadcasted_iota(jnp.int32, (B, npoint), 1)
    fi_ref[...] = jnp.zeros((B, npoint), jnp.int32)
    nx_ref[...] = jnp.zeros((B, npoint), jnp.float32)
    ny_ref[...] = jnp.zeros((B, npoint), jnp.float32)
    nz_ref[...] = jnp.zeros((B, npoint), jnp.float32)

    def step(k, carry):
        dist, far = carry
        oh = iota_n == far
        cx = jnp.sum(jnp.where(oh, x, 0.0), axis=1, keepdims=True)
        cy = jnp.sum(jnp.where(oh, y, 0.0), axis=1, keepdims=True)
        cz = jnp.sum(jnp.where(oh, z, 0.0), axis=1, keepdims=True)
        sel = iota_s == k
        fi_ref[...] = jnp.where(sel, far, fi_ref[...])
        nx_ref[...] = jnp.where(sel, cx, nx_ref[...])
        ny_ref[...] = jnp.where(sel, cy, ny_ref[...])
        nz_ref[...] = jnp.where(sel, cz, nz_ref[...])
        d = (x - cx) ** 2 + (y - cy) ** 2 + (z - cz) ** 2
        dist = jnp.minimum(dist, d)
        m = jnp.max(dist, axis=1, keepdims=True)
        nf = jnp.min(jnp.where(dist == m, iota_n, n), axis=1, keepdims=True)
        return dist, nf

    lax.fori_loop(0, npoint, step,
                  (jnp.full((B, n), 1e10, jnp.float32),
                   jnp.zeros((B, 1), jnp.int32)))


def _fps(x, y, z, npoint):
    n = x.shape[1]
    return pl.pallas_call(
        functools.partial(_fps_body, npoint, n),
        out_shape=[jax.ShapeDtypeStruct((B, npoint), jnp.int32)] +
                  [jax.ShapeDtypeStruct((B, npoint), jnp.float32)] * 3,
    )(x, y, z)


def _mlp_max_body(ch, k, nlayer, has_first, *refs):
    g_ref, ctr_ref, wc_ref = refs[0:3]
    pos = 3
    if has_first:
        w1_ref, b1_ref = refs[pos:pos + 2]
        pos += 2
    w_refs = refs[pos:pos + nlayer]
    b_refs = refs[pos + nlayer:pos + 2 * nlayer]
    o_ref = refs[pos + 2 * nlayer]
    x = g_ref[0]
    if has_first:
        x = (jnp.dot(x, w1_ref[...], preferred_element_type=jnp.float32)
             + b1_ref[...])
    h1 = x.shape[-1]
    proj = jnp.dot(ctr_ref[0], wc_ref[...], preferred_element_type=jnp.float32)
    x = jnp.maximum(x.reshape(ch, k, h1) - proj[:, None, :], 0.0)
    x = x.reshape(ch * k, h1)
    for i in range(nlayer):
        x = jnp.dot(x, w_refs[i][...], preferred_element_type=jnp.float32)
        x = jnp.maximum(x + b_refs[i][...], 0.0)
    c = x.shape[-1]
    o_ref[0] = jnp.max(x.reshape(ch, k, c), axis=1)


def _mlp_max(g, ctr, wc, ws, bs, s, k, first_w=None, first_b=None):
    cin = g.shape[-1]
    cout = ws[-1].shape[1]
    ch = max(1, 4096 // k)
    nch = s // ch
    nl = len(ws)
    has_first = first_w is not None
    in_specs = [pl.BlockSpec((1, ch * k, cin), lambda b, c: (b, c, 0)),
                pl.BlockSpec((1, ch, 3), lambda b, c: (b, c, 0)),
                pl.BlockSpec(wc.shape, lambda b, c: (0, 0))]
    args = [g, ctr, wc]
    if has_first:
        in_specs += [pl.BlockSpec(first_w.shape, lambda b, c: (0, 0)),
                     pl.BlockSpec(first_b.shape, lambda b, c: (0, 0))]
        args += [first_w, first_b]
    for w in ws:
        in_specs.append(pl.BlockSpec(w.shape, lambda b, c: (0, 0)))
    for bb in bs:
        in_specs.append(pl.BlockSpec(bb.shape, lambda b, c: (0, 0)))
    args += list(ws) + list(bs)
    return pl.pallas_call(
        functools.partial(_mlp_max_body, ch, k, nl, has_first),
        grid=(B, nch),
        in_specs=in_specs,
        out_specs=pl.BlockSpec((1, ch, cout), lambda b, c: (b, c, 0)),
        out_shape=jax.ShapeDtypeStruct((B, s, cout), jnp.float32),
    )(*args)


def _gsa_body(nlayer, *refs):
    x_ref = refs[0]
    w_refs = refs[1:1 + nlayer]
    b_refs = refs[1 + nlayer:1 + 2 * nlayer]
    o_ref = refs[1 + 2 * nlayer]
    x = x_ref[0]
    for i in range(nlayer):
        x = jnp.dot(x, w_refs[i][...], preferred_element_type=jnp.float32)
        x = jnp.maximum(x + b_refs[i][...], 0.0)
    o_ref[0] = jnp.max(x, axis=0, keepdims=True)


def _gsa(x, ws, bs):
    s, cin = x.shape[1], x.shape[2]
    cout = ws[-1].shape[1]
    nl = len(ws)
    in_specs = [pl.BlockSpec((1, s, cin), lambda b: (b, 0, 0))]
    in_specs += [pl.BlockSpec(w.shape, lambda b: (0, 0)) for w in ws]
    in_specs += [pl.BlockSpec(bb.shape, lambda b: (0, 0)) for bb in bs]
    return pl.pallas_call(
        functools.partial(_gsa_body, nl),
        grid=(B,),
        in_specs=in_specs,
        out_specs=pl.BlockSpec((1, 1, cout), lambda b: (b, 0, 0)),
        out_shape=jax.ShapeDtypeStruct((B, 1, cout), jnp.float32),
    )(x, *ws, *bs)


def _fp1_body(f_ref, g_ref, w1a_ref, w1b_ref, b1_ref, w2_ref, b2_ref, o_ref):
    f = f_ref[0]
    gtop = jnp.dot(g_ref[0], w1b_ref[...], preferred_element_type=jnp.float32)
    h = jnp.dot(f, w1a_ref[...], preferred_element_type=jnp.float32)
    h = jnp.maximum(h + gtop + b1_ref[...], 0.0)
    h = jnp.dot(h, w2_ref[...], preferred_element_type=jnp.float32)
    o_ref[0] = jnp.maximum(h + b2_ref[...], 0.0)


def _fp1(feats1, gvec, w1, b1, w2, b2):
    s, c1 = feats1.shape[1], feats1.shape[2]
    cg = gvec.shape[-1]
    w1a, w1b = w1[:c1], w1[c1:]
    cout = w2.shape[1]
    return pl.pallas_call(
        _fp1_body,
        grid=(B,),
        in_specs=[
            pl.BlockSpec((1, s, c1), lambda b: (b, 0, 0)),
            pl.BlockSpec((1, 1, cg), lambda b: (b, 0, 0)),
            pl.BlockSpec(w1a.shape, lambda b: (0, 0)),
            pl.BlockSpec(w1b.shape, lambda b: (0, 0)),
            pl.BlockSpec(b1.shape, lambda b: (0, 0)),
            pl.BlockSpec(w2.shape, lambda b: (0, 0)),
            pl.BlockSpec(b2.shape, lambda b: (0, 0)),
        ],
        out_specs=pl.BlockSpec((1, s, cout), lambda b: (b, 0, 0)),
        out_shape=jax.ShapeDtypeStruct((B, s, cout), jnp.float32),
    )(feats1, gvec, w1a, w1b, b1, w2, b2)


def _interp_w(p1, p2t):
    d = ((p1[:, 0:1] - p2t[0:1, :]) ** 2
         + (p1[:, 1:2] - p2t[1:2, :]) ** 2
         + (p1[:, 2:3] - p2t[2:3, :]) ** 2)
    big = jnp.float32(3e38)
    t = d
    m1 = jnp.min(t, axis=1, keepdims=True)
    t = jnp.where(t == m1, big, t)
    m2 = jnp.min(t, axis=1, keepdims=True)
    t = jnp.where(t == m2, big, t)
    m3 = jnp.min(t, axis=1, keepdims=True)
    mask = d <= m3
    recip = jnp.where(mask, 1.0 / (d + 1e-8), 0.0)
    return recip / jnp.sum(recip, axis=1, keepdims=True)


def _fp2_body(p1_ref, p2t_ref, f1_ref, f2_ref,
              w1a_ref, w1b_ref, b1_ref, w2_ref, b2_ref, o_ref):
    w = _interp_w(p1_ref[0], p2t_ref[0])
    interp = jnp.dot(w, f2_ref[0], preferred_element_type=jnp.float32)
    h = (jnp.dot(f1_ref[0], w1a_ref[...], preferred_element_type=jnp.float32)
         + jnp.dot(interp, w1b_ref[...], preferred_element_type=jnp.float32))
    h = jnp.maximum(h + b1_ref[...], 0.0)
    h = jnp.dot(h, w2_ref[...], preferred_element_type=jnp.float32)
    o_ref[0] = jnp.maximum(h + b2_ref[...], 0.0)


def _fp2(p1, p2t, feats1, feats2, w1, b1, w2, b2):
    s1, s2 = p1.shape[1], p2t.shape[2]
    c1, c2 = feats1.shape[2], feats2.shape[2]
    w1a, w1b = w1[:c1], w1[c1:]
    cout = w2.shape[1]
    return pl.pallas_call(
        _fp2_body,
        grid=(B,),
        in_specs=[
            pl.BlockSpec((1, s1, 3), lambda b: (b, 0, 0)),
            pl.BlockSpec((1, 3, s2), lambda b: (b, 0, 0)),
            pl.BlockSpec((1, s1, c1), lambda b: (b, 0, 0)),
            pl.BlockSpec((1, s2, c2), lambda b: (b, 0, 0)),
            pl.BlockSpec(w1a.shape, lambda b: (0, 0)),
            pl.BlockSpec(w1b.shape, lambda b: (0, 0)),
            pl.BlockSpec(b1.shape, lambda b: (0, 0)),
            pl.BlockSpec(w2.shape, lambda b: (0, 0)),
            pl.BlockSpec(b2.shape, lambda b: (0, 0)),
        ],
        out_specs=pl.BlockSpec((1, s1, cout), lambda b: (b, 0, 0)),
        out_shape=jax.ShapeDtypeStruct((B, s1, cout), jnp.float32),
    )(p1, p2t, feats1, feats2, w1a, w1b, b1, w2, b2)


def _fp3_head(p1, p2t, oh, f0t, feats2, w1, b1, w2, b2, wc1, bc1, wc2, bc2):
    s1, s2 = p1.shape[1], p2t.shape[2]
    c2 = feats2.shape[2]
    woh, wf, wx, wi = (w1[:NUM_CLASS_K], w1[NUM_CLASS_K:NUM_CLASS_K + 3],
                       w1[NUM_CLASS_K + 3:NUM_CLASS_K + 6],
                       w1[NUM_CLASS_K + 6:])
    npart = wc2.shape[1]

    def body(p1_ref, p2t_ref, oh_ref, f0_ref, f2_ref,
             woh_ref, wf_ref, wx_ref, wi_ref, b1_ref, w2_ref, b2_ref,
             wc1_ref, bc1_ref, wc2_ref, bc2_ref, o_ref):
        w = _interp_w(p1_ref[0], p2t_ref[0])
        interp = jnp.dot(w, f2_ref[0], preferred_element_type=jnp.float32)
        ohrow = jnp.dot(oh_ref[0], woh_ref[...],
                        preferred_element_type=jnp.float32)
        h = (jnp.dot(f0_ref[0], wf_ref[...], preferred_element_type=jnp.float32)
             + jnp.dot(p1_ref[0], wx_ref[...], preferred_element_type=jnp.float32)
             + jnp.dot(interp, wi_ref[...], preferred_element_type=jnp.float32))
        h = jnp.maximum(h + ohrow + b1_ref[...], 0.0)
        h = jnp.maximum(jnp.dot(h, w2_ref[...],
                                preferred_element_type=jnp.float32)
                        + b2_ref[...], 0.0)
        h = jnp.maximum(jnp.dot(h, wc1_ref[...],
                                preferred_element_type=jnp.float32)
                        + bc1_ref[...], 0.0)
        out = lax.dot_general(wc2_ref[...], h, (((0,), (1,)), ((), ())),
                              preferred_element_type=jnp.float32)
        o_ref[0] = out + bc2_ref[...].reshape(npart, 1)

    return pl.pallas_call(
        body,
        grid=(B,),
        in_specs=[
            pl.BlockSpec((1, s1, 3), lambda b: (b, 0, 0)),
            pl.BlockSpec((1, 3, s2), lambda b: (b, 0, 0)),
            pl.BlockSpec((1, 1, NUM_CLASS_K), lambda b: (b, 0, 0)),
            pl.BlockSpec((1, s1, 3), lambda b: (b, 0, 0)),
            pl.BlockSpec((1, s2, c2), lambda b: (b, 0, 0)),
            pl.BlockSpec(woh.shape, lambda b: (0, 0)),
            pl.BlockSpec(wf.shape, lambda b: (0, 0)),
            pl.BlockSpec(wx.shape, lambda b: (0, 0)),
            pl.BlockSpec(wi.shape, lambda b: (0, 0)),
            pl.BlockSpec(b1.shape, lambda b: (0, 0)),
            pl.BlockSpec(w2.shape, lambda b: (0, 0)),
            pl.BlockSpec(b2.shape, lambda b: (0, 0)),
            pl.BlockSpec(wc1.shape, lambda b: (0, 0)),
            pl.BlockSpec(bc1.shape, lambda b: (0, 0)),
            pl.BlockSpec(wc2.shape, lambda b: (0, 0)),
            pl.BlockSpec(bc2.shape, lambda b: (0, 0)),
        ],
        out_specs=pl.BlockSpec((1, npart, s1), lambda b: (b, 0, 0)),
        out_shape=jax.ShapeDtypeStruct((B, npart, s1), jnp.float32),
    )(p1, p2t, oh, f0t, feats2, woh, wf, wx, wi, b1, w2, b2,
      wc1, bc1, wc2, bc2)


def _ballq_sc(px, py, pz, cx, cy, cz, radii, ks):
    n = px.shape[1]
    s = cx.shape[1]
    nw = 32
    per = (B * s) // nw
    nch = n // 16
    nbr = len(radii)
    r2 = [jnp.float32(r * r) for r in radii]
    mesh = plsc.VectorSubcoreMesh(core_axis_name="c", subcore_axis_name="s")

    def body(px_h, py_h, pz_h, cx_h, cy_h, cz_h, *rest):
        iota = lax.iota(jnp.int32, 16)
        out_hs = rest[:nbr]
        xs, ys, zs, cxs, cys, czs = rest[nbr:nbr + 6]
        bufs = rest[nbr + 6:nbr + 6 + nbr]
        cbufs = rest[nbr + 6 + nbr:nbr + 6 + 2 * nbr]
        shs = rest[nbr + 6 + 2 * nbr]
        w = lax.axis_index("s") * 2 + lax.axis_index("c")
        b = (w * per) // s
        lo = (w * per) % s
        pltpu.sync_copy(px_h.at[b], xs)
        pltpu.sync_copy(py_h.at[b], ys)
        pltpu.sync_copy(pz_h.at[b], zs)
        pltpu.sync_copy(cx_h.at[b, pl.ds(lo, per)], cxs.at[pl.ds(0, per)])
        pltpu.sync_copy(cy_h.at[b, pl.ds(lo, per)], cys.at[pl.ds(0, per)])
        pltpu.sync_copy(cz_h.at[b, pl.ds(lo, per)], czs.at[pl.ds(0, per)])

        for a in range(3):
            shs[pl.ds(a * 48, 16)] = jnp.zeros((16,), jnp.int32)
            shs[pl.ds(a * 48 + 32, 16)] = jnp.zeros((16,), jnp.int32)

        def psum16(r, a):
            for t in (1, 2, 4, 8):
                shs[pl.ds(a * 48 + 16, 16)] = r
                r = r + shs[pl.ds(a * 48 + 16 - t, 16)]
            return r

        def compact16(v, disp):
            for t in (1, 2, 4, 8):
                shs[pl.ds(48 + 16, 16)] = v
                shs[pl.ds(96 + 16, 16)] = disp
                vsh = shs[pl.ds(48 + 16 + t, 16)]
                dsh = shs[pl.ds(96 + 16 + t, 16)]
                cond = (dsh & t) != 0
                v = jnp.where(cond, vsh, v)
                disp = jnp.where(cond, dsh - t, disp)
            return v

        def center_step(i, _):
            cxi = cxs[pl.ds(i, 16)][0]
            cyi = cys[pl.ds(i, 16)][0]
            czi = czs[pl.ds(i, 16)][0]

            def chunk_step(c, os_):
                xv = xs[pl.ds(c * 16, 16)]
                yv = ys[pl.ds(c * 16, 16)]
                zv = zs[pl.ds(c * 16, 16)]
                dx = xv - cxi
                dy = yv - cyi
                dz = zv - czi
                d = dx * dx + dy * dy + dz * dz
                idxv = c * 16 + iota
                new_os = []
                for j in range(nbr):
                    k = ks[j]
                    oj = os_[j]
                    m = d <= r2[j]

                    def do(o, m=m, j=j, k=k, idxv=idxv):
                        mi = jnp.where(m, 1, 0)
                        r = psum16(mi, 0)
                        cnt = r[15]

                        def wr(o, m=m, j=j, k=k, idxv=idxv, r=r, cnt=cnt):
                            disp = jnp.where(m, iota - (r - 1), 0)
                            v = compact16(idxv, disp)
                            bufs[j][pl.ds(i * (k + 16) + o, 16)] = v
                            return jnp.minimum(o + cnt, k)

                        return lax.cond(cnt > 0, wr, lambda o: o, o)

                    new_os.append(lax.cond(oj < k, do, lambda o: o, oj))
                return tuple(new_os)

            os_ = lax.fori_loop(0, nch, chunk_step,
                                tuple(jnp.int32(0) for _ in range(nbr)))
            for j in range(nbr):
                k = ks[j]
                first = bufs[j][pl.ds(i * (k + 16), 16)][0]
                for kc in range(k // 16):
                    pos = kc * 16 + iota
                    cur = bufs[j][pl.ds(i * (k + 16) + kc * 16, 16)]
                    cbufs[j][pl.ds(i * k + kc * 16, 16)] = jnp.where(
                        pos >= os_[j], first, cur)
            return 0

        lax.fori_loop(0, per, center_step, 0)
        for j in range(nbr):
            k = ks[j]
            pltpu.sync_copy(cbufs[j],
                            out_hs[j].at[pl.ds((b * s + lo) * k, per * k)])

    fn = pl.kernel(
        body,
        out_type=[jax.ShapeDtypeStruct((B * s * k,), jnp.int32) for k in ks],
        mesh=mesh,
        scratch_types=(
            [pltpu.VMEM((n,), jnp.float32)] * 3
            + [pltpu.VMEM((per + 16,), jnp.float32)] * 3
            + [pltpu.VMEM((per * (k + 16),), jnp.int32) for k in ks]
            + [pltpu.VMEM((per * k,), jnp.int32) for k in ks]
            + [pltpu.VMEM((144,), jnp.int32)]),
    )
    outs = fn(px, py, pz, cx, cy, cz)
    return [o.reshape(B, s, k) for o, k in zip(outs, ks)]


def _gather_sc(table, idx):
    bb, n, d = table.shape
    m = idx.shape[1]
    nw = 32
    rpw = (bb * m) // nw
    nchunk = rpw // 128
    mesh = plsc.VectorSubcoreMesh(core_axis_name="c", subcore_axis_name="s")
    idx_flat = idx.reshape(bb * m)

    def body(tab_h, idx_h, out_h, idx_v, rows_v, sem):
        w = lax.axis_index("s") * 2 + lax.axis_index("c")
        base = w * rpw
        b = base // m

        def chunk(c, _):
            off = base + c * 128
            pltpu.sync_copy(idx_h.at[pl.ds(off, 128)], idx_v)
            pltpu.async_copy(tab_h.at[b].at[idx_v], rows_v, sem).wait()
            pltpu.sync_copy(rows_v, out_h.at[pl.ds(off, 128)])
            return 0

        lax.fori_loop(0, nchunk, chunk, 0)

    fn = pl.kernel(
        body,
        out_type=jax.ShapeDtypeStruct((bb * m, d), jnp.float32),
        mesh=mesh,
        scratch_types=[
            pltpu.VMEM((128,), jnp.int32),
            pltpu.VMEM((128, d), jnp.float32),
            pltpu.SemaphoreType.DMA,
        ],
    )
    return fn(table, idx_flat).reshape(bb, m, d)


def _dense_body(x_ref, w_ref, b_ref, o_ref):
    o_ref[0] = (jnp.dot(x_ref[0], w_ref[...],
                        preferred_element_type=jnp.float32) + b_ref[...])


def _dense(x, w, b):
    s, cin = x.shape[1], x.shape[2]
    cout = w.shape[1]
    return pl.pallas_call(
        _dense_body,
        grid=(B,),
        in_specs=[
            pl.BlockSpec((1, s, cin), lambda bi: (bi, 0, 0)),
            pl.BlockSpec(w.shape, lambda bi: (0, 0)),
            pl.BlockSpec(b.shape, lambda bi: (0, 0)),
        ],
        out_specs=pl.BlockSpec((1, s, cout), lambda bi: (bi, 0, 0)),
        out_shape=jax.ShapeDtypeStruct((B, s, cout), jnp.float32),
    )(x, w, b)


def _ball_glue(r, k, xyz, new_xyz):
    n = xyz.shape[1]
    sqr = jnp.sum((new_xyz[:, :, None, :] - xyz[:, None, :, :]) ** 2, axis=-1)
    gid = jnp.where(sqr > r * r, n,
                    jnp.arange(n, dtype=jnp.int32)[None, None, :])
    gid = jnp.sort(gid, axis=-1)[:, :, :k]
    first = gid[:, :, 0:1]
    return jnp.where(gid == n, first, gid)


def _gather_pts(pts, idx):
    bidx = jnp.arange(pts.shape[0]).reshape((-1,) + (1,) * (idx.ndim - 1))
    return pts[bidx, idx]


def kernel(points, features, class_ids, params):
    x, y, z = points[:, 0], points[:, 1], points[:, 2]
    f0t = jnp.transpose(features, (0, 2, 1))
    xyz = jnp.stack([x, y, z], axis=-1)
    oh = jax.nn.one_hot(class_ids, NUM_CLASS_K, dtype=jnp.float32)[:, None, :]

    ms1 = [[_fold_cbr(p) for p in mlp] for mlp in params['ms1']]
    ms2 = [[_fold_cbr(p) for p in mlp] for mlp in params['ms2']]
    gsa_p = [_fold_cbr(p) for p in params['gsa']]
    fp1_p = [_fold_cbr(p) for p in params['fp1']]
    fp2_p = [_fold_cbr(p) for p in params['fp2']]
    fp3_p = [_fold_cbr(p) for p in params['fp3']]
    wc1, bc1 = _fold_cbr(params['cls1'])
    wc2 = params['cls2']['W']
    bc2 = params['cls2']['b'][None, :]

    _, nx, ny, nz = _fps(x, y, z, 512)
    l1_xyz = jnp.stack([nx, ny, nz], axis=-1)
    idxs1 = _ballq_sc(x, y, z, nx, ny, nz, [0.1, 0.2, 0.4], [32, 64, 128])
    t8 = jnp.concatenate([f0t, xyz, jnp.zeros((B, N0, 2), jnp.float32)],
                         axis=-1)
    outs = []
    for idx, k, mlp in zip(idxs1, [32, 64, 128], ms1):
        w1 = mlp[0][0]
        h1 = w1.shape[1]
        w1p = _padrc(w1, 8, 128)
        p1 = _dense(t8, w1p, _padc(mlp[0][1], 128))
        g = _gather_sc(p1, idx.reshape(B, 512 * k))
        w2p = _padrc(mlp[1][0], 128, mlp[1][0].shape[1])
        outs.append(_mlp_max(g, l1_xyz, _padc(w1[3:6], 128),
                             [w2p, mlp[2][0]], [mlp[1][1], mlp[2][1]],
                             512, k))
    l1_f = jnp.concatenate(outs, axis=-1)

    _, nx2, ny2, nz2 = _fps(nx, ny, nz, 128)
    l2_xyz = jnp.stack([nx2, ny2, nz2], axis=-1)
    l1_cat = jnp.concatenate(
        [l1_f, l1_xyz, jnp.zeros((B, 512, 61), jnp.float32)],
        axis=-1)
    idxs2 = _ballq_sc(nx, ny, nz, nx2, ny2, nz2, [0.4, 0.8], [64, 128])
    outs2 = []
    for idx, k, mlp in zip(idxs2, [64, 128], ms2):
        w1 = mlp[0][0]
        w1p = jnp.concatenate([w1, jnp.zeros((61, w1.shape[1]), jnp.float32)],
                              axis=0)
        proj1 = _dense(l1_cat, w1p, mlp[0][1])
        gp = _gather_sc(proj1, idx.reshape(B, 128 * k))
        outs2.append(_mlp_max(gp, l2_xyz, w1[320:323],
                              [w for w, _ in mlp[1:]],
                              [b for _, b in mlp[1:]], 128, k))
    l2_f = jnp.concatenate(outs2, axis=-1)

    x2 = jnp.concatenate([l2_f, l2_xyz], axis=-1)
    gvec = _gsa(x2, [w for w, _ in gsa_p], [b for _, b in gsa_p])

    l2_xyzt = jnp.stack([nx2, ny2, nz2], axis=1)
    l1_xyzt = jnp.stack([nx, ny, nz], axis=1)
    l2_fn = _fp1(l2_f, gvec, fp1_p[0][0], fp1_p[0][1],
                 fp1_p[1][0], fp1_p[1][1])
    l1_fn = _fp2(l1_xyz, l2_xyzt, l1_f, l2_fn,
                 fp2_p[0][0], fp2_p[0][1], fp2_p[1][0], fp2_p[1][1])
    out = _fp3_head(xyz, l1_xyzt, oh, f0t, l1_fn,
                    fp3_p[0][0], fp3_p[0][1], fp3_p[1][0], fp3_p[1][1],
                    wc1, bc1, wc2, bc2)
    return out


# speedup vs baseline: 15.4791x; 1.2792x over previous
import functools

import jax
import jax.numpy as jnp
from jax import lax
from jax.experimental import pallas as pl
from jax.experimental.pallas import tpu as pltpu
from jax.experimental.pallas import tpu_sc as plsc

NUM_CLASS_K = 16
B = 8
N0 = 2048


def _fold_cbr(p):
    s = p['g'] / jnp.sqrt(1.0 + 1e-5)
    return p['W'] * s[None, :], (p['b'] * s + p['be'])[None, :]


def _padc(a, n):
    return jnp.concatenate(
        [a, jnp.zeros(a.shape[:-1] + (n - a.shape[-1],), jnp.float32)],
        axis=-1)


def _padrc(a, nr, nc):
    a = jnp.concatenate(
        [a, jnp.zeros((nr - a.shape[0], a.shape[1]), jnp.float32)], axis=0)
    return _padc(a, nc)


def _fps_body(npoint, n, x_ref, y_ref, z_ref, fi_ref, nx_ref, ny_ref, nz_ref):
    x = x_ref[...]
    y = y_ref[...]
    z = z_ref[...]
    iota_n = lax.broadcasted_iota(jnp.int32, (B, n), 1)
    iota_s = lax.broadcasted_iota(jnp.int32, (B, npoint), 1)
    fi_ref[...] = jnp.zeros((B, npoint), jnp.int32)
    nx_ref[...] = jnp.zeros((B, npoint), jnp.float32)
    ny_ref[...] = jnp.zeros((B, npoint), jnp.float32)
    nz_ref[...] = jnp.zeros((B, npoint), jnp.float32)

    def step(k, carry):
        dist, far = carry
        oh = iota_n == far
        cx = jnp.sum(jnp.where(oh, x, 0.0), axis=1, keepdims=True)
        cy = jnp.sum(jnp.where(oh, y, 0.0), axis=1, keepdims=True)
        cz = jnp.sum(jnp.where(oh, z, 0.0), axis=1, keepdims=True)
        sel = iota_s == k
        fi_ref[...] = jnp.where(sel, far, fi_ref[...])
        nx_ref[...] = jnp.where(sel, cx, nx_ref[...])
        ny_ref[...] = jnp.where(sel, cy, ny_ref[...])
        nz_ref[...] = jnp.where(sel, cz, nz_ref[...])
        d = (x - cx) ** 2 + (y - cy) ** 2 + (z - cz) ** 2
        dist = jnp.minimum(dist, d)
        m = jnp.max(dist, axis=1, keepdims=True)
        nf = jnp.min(jnp.where(dist == m, iota_n, n), axis=1, keepdims=True)
        return dist, nf

    lax.fori_loop(0, npoint, step,
                  (jnp.full((B, n), 1e10, jnp.float32),
                   jnp.zeros((B, 1), jnp.int32)))


def _fps(x, y, z, npoint):
    n = x.shape[1]
    return pl.pallas_call(
        functools.partial(_fps_body, npoint, n),
        out_shape=[jax.ShapeDtypeStruct((B, npoint), jnp.int32)] +
                  [jax.ShapeDtypeStruct((B, npoint), jnp.float32)] * 3,
    )(x, y, z)


def _mlp_max_body(ch, k, nlayer, has_first, *refs):
    g_ref, ctr_ref, wc_ref = refs[0:3]
    pos = 3
    if has_first:
        w1_ref, b1_ref = refs[pos:pos + 2]
        pos += 2
    w_refs = refs[pos:pos + nlayer]
    b_refs = refs[pos + nlayer:pos + 2 * nlayer]
    o_ref = refs[pos + 2 * nlayer]
    x = g_ref[0]
    if has_first:
        x = (jnp.dot(x, w1_ref[...], preferred_element_type=jnp.float32)
             + b1_ref[...])
    h1 = x.shape[-1]
    proj = jnp.dot(ctr_ref[0], wc_ref[...], preferred_element_type=jnp.float32)
    x = jnp.maximum(x.reshape(ch, k, h1) - proj[:, None, :], 0.0)
    x = x.reshape(ch * k, h1)
    for i in range(nlayer):
        x = jnp.dot(x, w_refs[i][...], preferred_element_type=jnp.float32)
        x = jnp.maximum(x + b_refs[i][...], 0.0)
    c = x.shape[-1]
    o_ref[0] = jnp.max(x.reshape(ch, k, c), axis=1)


def _mlp_max(g, ctr, wc, ws, bs, s, k, first_w=None, first_b=None):
    cin = g.shape[-1]
    cout = ws[-1].shape[1]
    ch = max(1, 4096 // k)
    nch = s // ch
    nl = len(ws)
    has_first = first_w is not None
    in_specs = [pl.BlockSpec((1, ch * k, cin), lambda b, c: (b, c, 0)),
                pl.BlockSpec((1, ch, 3), lambda b, c: (b, c, 0)),
                pl.BlockSpec(wc.shape, lambda b, c: (0, 0))]
    args = [g, ctr, wc]
    if has_first:
        in_specs += [pl.BlockSpec(first_w.shape, lambda b, c: (0, 0)),
                     pl.BlockSpec(first_b.shape, lambda b, c: (0, 0))]
        args += [first_w, first_b]
    for w in ws:
        in_specs.append(pl.BlockSpec(w.shape, lambda b, c: (0, 0)))
    for bb in bs:
        in_specs.append(pl.BlockSpec(bb.shape, lambda b, c: (0, 0)))
    args += list(ws) + list(bs)
    return pl.pallas_call(
        functools.partial(_mlp_max_body, ch, k, nl, has_first),
        grid=(B, nch),
        in_specs=in_specs,
        out_specs=pl.BlockSpec((1, ch, cout), lambda b, c: (b, c, 0)),
        out_shape=jax.ShapeDtypeStruct((B, s, cout), jnp.float32),
    )(*args)


def _gsa_body(nlayer, *refs):
    x_ref = refs[0]
    w_refs = refs[1:1 + nlayer]
    b_refs = refs[1 + nlayer:1 + 2 * nlayer]
    o_ref = refs[1 + 2 * nlayer]
    x = x_ref[0]
    for i in range(nlayer):
        x = jnp.dot(x, w_refs[i][...], preferred_element_type=jnp.float32)
        x = jnp.maximum(x + b_refs[i][...], 0.0)
    o_ref[0] = jnp.max(x, axis=0, keepdims=True)


def _gsa(x, ws, bs):
    s, cin = x.shape[1], x.shape[2]
    cout = ws[-1].shape[1]
    nl = len(ws)
    in_specs = [pl.BlockSpec((1, s, cin), lambda b: (b, 0, 0))]
    in_specs += [pl.BlockSpec(w.shape, lambda b: (0, 0)) for w in ws]
    in_specs += [pl.BlockSpec(bb.shape, lambda b: (0, 0)) for bb in bs]
    return pl.pallas_call(
        functools.partial(_gsa_body, nl),
        grid=(B,),
        in_specs=in_specs,
        out_specs=pl.BlockSpec((1, 1, cout), lambda b: (b, 0, 0)),
        out_shape=jax.ShapeDtypeStruct((B, 1, cout), jnp.float32),
    )(x, *ws, *bs)


def _fp1_body(f_ref, g_ref, w1a_ref, w1b_ref, b1_ref, w2_ref, b2_ref, o_ref):
    f = f_ref[0]
    gtop = jnp.dot(g_ref[0], w1b_ref[...], preferred_element_type=jnp.float32)
    h = jnp.dot(f, w1a_ref[...], preferred_element_type=jnp.float32)
    h = jnp.maximum(h + gtop + b1_ref[...], 0.0)
    h = jnp.dot(h, w2_ref[...], preferred_element_type=jnp.float32)
    o_ref[0] = jnp.maximum(h + b2_ref[...], 0.0)


def _fp1(feats1, gvec, w1, b1, w2, b2):
    s, c1 = feats1.shape[1], feats1.shape[2]
    cg = gvec.shape[-1]
    w1a, w1b = w1[:c1], w1[c1:]
    cout = w2.shape[1]
    return pl.pallas_call(
        _fp1_body,
        grid=(B,),
        in_specs=[
            pl.BlockSpec((1, s, c1), lambda b: (b, 0, 0)),
            pl.BlockSpec((1, 1, cg), lambda b: (b, 0, 0)),
            pl.BlockSpec(w1a.shape, lambda b: (0, 0)),
            pl.BlockSpec(w1b.shape, lambda b: (0, 0)),
            pl.BlockSpec(b1.shape, lambda b: (0, 0)),
            pl.BlockSpec(w2.shape, lambda b: (0, 0)),
            pl.BlockSpec(b2.shape, lambda b: (0, 0)),
        ],
        out_specs=pl.BlockSpec((1, s, cout), lambda b: (b, 0, 0)),
        out_shape=jax.ShapeDtypeStruct((B, s, cout), jnp.float32),
    )(feats1, gvec, w1a, w1b, b1, w2, b2)


def _interp_w(p1, p2t):
    d = ((p1[:, 0:1] - p2t[0:1, :]) ** 2
         + (p1[:, 1:2] - p2t[1:2, :]) ** 2
         + (p1[:, 2:3] - p2t[2:3, :]) ** 2)
    big = jnp.float32(3e38)
    t = d
    m1 = jnp.min(t, axis=1, keepdims=True)
    t = jnp.where(t == m1, big, t)
    m2 = jnp.min(t, axis=1, keepdims=True)
    t = jnp.where(t == m2, big, t)
    m3 = jnp.min(t, axis=1, keepdims=True)
    mask = d <= m3
    recip = jnp.where(mask, 1.0 / (d + 1e-8), 0.0)
    return recip / jnp.sum(recip, axis=1, keepdims=True)


def _fp2_body(p1_ref, p2t_ref, f1_ref, f2_ref,
              w1a_ref, w1b_ref, b1_ref, w2_ref, b2_ref, o_ref):
    w = _interp_w(p1_ref[0], p2t_ref[0])
    interp = jnp.dot(w, f2_ref[0], preferred_element_type=jnp.float32)
    h = (jnp.dot(f1_ref[0], w1a_ref[...], preferred_element_type=jnp.float32)
         + jnp.dot(interp, w1b_ref[...], preferred_element_type=jnp.float32))
    h = jnp.maximum(h + b1_ref[...], 0.0)
    h = jnp.dot(h, w2_ref[...], preferred_element_type=jnp.float32)
    o_ref[0] = jnp.maximum(h + b2_ref[...], 0.0)


def _fp2(p1, p2t, feats1, feats2, w1, b1, w2, b2):
    s1, s2 = p1.shape[1], p2t.shape[2]
    c1, c2 = feats1.shape[2], feats2.shape[2]
    w1a, w1b = w1[:c1], w1[c1:]
    cout = w2.shape[1]
    return pl.pallas_call(
        _fp2_body,
        grid=(B,),
        in_specs=[
            pl.BlockSpec((1, s1, 3), lambda b: (b, 0, 0)),
            pl.BlockSpec((1, 3, s2), lambda b: (b, 0, 0)),
            pl.BlockSpec((1, s1, c1), lambda b: (b, 0, 0)),
            pl.BlockSpec((1, s2, c2), lambda b: (b, 0, 0)),
            pl.BlockSpec(w1a.shape, lambda b: (0, 0)),
            pl.BlockSpec(w1b.shape, lambda b: (0, 0)),
            pl.BlockSpec(b1.shape, lambda b: (0, 0)),
            pl.BlockSpec(w2.shape, lambda b: (0, 0)),
            pl.BlockSpec(b2.shape, lambda b: (0, 0)),
        ],
        out_specs=pl.BlockSpec((1, s1, cout), lambda b: (b, 0, 0)),
        out_shape=jax.ShapeDtypeStruct((B, s1, cout), jnp.float32),
    )(p1, p2t, feats1, feats2, w1a, w1b, b1, w2, b2)


def _fp3_head(p1, p2t, oh, f0t, feats2, w1, b1, w2, b2, wc1, bc1, wc2, bc2):
    s1, s2 = p1.shape[1], p2t.shape[2]
    c2 = feats2.shape[2]
    woh, wf, wx, wi = (w1[:NUM_CLASS_K], w1[NUM_CLASS_K:NUM_CLASS_K + 3],
                       w1[NUM_CLASS_K + 3:NUM_CLASS_K + 6],
                       w1[NUM_CLASS_K + 6:])
    npart = wc2.shape[1]

    def body(p1_ref, p2t_ref, oh_ref, f0_ref, f2_ref,
             woh_ref, wf_ref, wx_ref, wi_ref, b1_ref, w2_ref, b2_ref,
             wc1_ref, bc1_ref, wc2_ref, bc2_ref, o_ref):
        w = _interp_w(p1_ref[0], p2t_ref[0])
        interp = jnp.dot(w, f2_ref[0], preferred_element_type=jnp.float32)
        ohrow = jnp.dot(oh_ref[0], woh_ref[...],
                        preferred_element_type=jnp.float32)
        h = (jnp.dot(f0_ref[0], wf_ref[...], preferred_element_type=jnp.float32)
             + jnp.dot(p1_ref[0], wx_ref[...], preferred_element_type=jnp.float32)
             + jnp.dot(interp, wi_ref[...], preferred_element_type=jnp.float32))
        h = jnp.maximum(h + ohrow + b1_ref[...], 0.0)
        h = jnp.maximum(jnp.dot(h, w2_ref[...],
                                preferred_element_type=jnp.float32)
                        + b2_ref[...], 0.0)
        h = jnp.maximum(jnp.dot(h, wc1_ref[...],
                                preferred_element_type=jnp.float32)
                        + bc1_ref[...], 0.0)
        out = lax.dot_general(wc2_ref[...], h, (((0,), (1,)), ((), ())),
                              preferred_element_type=jnp.float32)
        o_ref[0] = out + bc2_ref[...].reshape(npart, 1)

    return pl.pallas_call(
        body,
        grid=(B,),
        in_specs=[
            pl.BlockSpec((1, s1, 3), lambda b: (b, 0, 0)),
            pl.BlockSpec((1, 3, s2), lambda b: (b, 0, 0)),
            pl.BlockSpec((1, 1, NUM_CLASS_K), lambda b: (b, 0, 0)),
            pl.BlockSpec((1, s1, 3), lambda b: (b, 0, 0)),
            pl.BlockSpec((1, s2, c2), lambda b: (b, 0, 0)),
            pl.BlockSpec(woh.shape, lambda b: (0, 0)),
            pl.BlockSpec(wf.shape, lambda b: (0, 0)),
            pl.BlockSpec(wx.shape, lambda b: (0, 0)),
            pl.BlockSpec(wi.shape, lambda b: (0, 0)),
            pl.BlockSpec(b1.shape, lambda b: (0, 0)),
            pl.BlockSpec(w2.shape, lambda b: (0, 0)),
            pl.BlockSpec(b2.shape, lambda b: (0, 0)),
            pl.BlockSpec(wc1.shape, lambda b: (0, 0)),
            pl.BlockSpec(bc1.shape, lambda b: (0, 0)),
            pl.BlockSpec(wc2.shape, lambda b: (0, 0)),
            pl.BlockSpec(bc2.shape, lambda b: (0, 0)),
        ],
        out_specs=pl.BlockSpec((1, npart, s1), lambda b: (b, 0, 0)),
        out_shape=jax.ShapeDtypeStruct((B, npart, s1), jnp.float32),
    )(p1, p2t, oh, f0t, feats2, woh, wf, wx, wi, b1, w2, b2,
      wc1, bc1, wc2, bc2)


def _ballq_sc(px, py, pz, cx, cy, cz, radii, ks):
    n = px.shape[1]
    s = cx.shape[1]
    nw = 32
    per = (B * s) // nw
    nch = n // 16
    nbr = len(radii)
    r2 = [jnp.float32(r * r) for r in radii]
    mesh = plsc.VectorSubcoreMesh(core_axis_name="c", subcore_axis_name="s")

    def body(px_h, py_h, pz_h, cx_h, cy_h, cz_h, *rest):
        iota = lax.iota(jnp.int32, 16)
        out_hs = rest[:nbr]
        xs, ys, zs, cxs, cys, czs = rest[nbr:nbr + 6]
        bufs = rest[nbr + 6:nbr + 6 + nbr]
        cbufs = rest[nbr + 6 + nbr:nbr + 6 + 2 * nbr]
        shs = rest[nbr + 6 + 2 * nbr]
        w = lax.axis_index("s") * 2 + lax.axis_index("c")
        b = (w * per) // s
        lo = (w * per) % s
        pltpu.sync_copy(px_h.at[b], xs)
        pltpu.sync_copy(py_h.at[b], ys)
        pltpu.sync_copy(pz_h.at[b], zs)
        pltpu.sync_copy(cx_h.at[b, pl.ds(lo, per)], cxs.at[pl.ds(0, per)])
        pltpu.sync_copy(cy_h.at[b, pl.ds(lo, per)], cys.at[pl.ds(0, per)])
        pltpu.sync_copy(cz_h.at[b, pl.ds(lo, per)], czs.at[pl.ds(0, per)])

        for a in range(3):
            shs[pl.ds(a * 48, 16)] = jnp.zeros((16,), jnp.int32)
            shs[pl.ds(a * 48 + 32, 16)] = jnp.zeros((16,), jnp.int32)

        def psum16(r, a):
            for t in (1, 2, 4, 8):
                shs[pl.ds(a * 48 + 16, 16)] = r
                r = r + shs[pl.ds(a * 48 + 16 - t, 16)]
            return r

        def compact16(v, disp):
            for t in (1, 2, 4, 8):
                shs[pl.ds(48 + 16, 16)] = v
                shs[pl.ds(96 + 16, 16)] = disp
                vsh = shs[pl.ds(48 + 16 + t, 16)]
                dsh = shs[pl.ds(96 + 16 + t, 16)]
                cond = (dsh & t) != 0
                v = jnp.where(cond, vsh, v)
                disp = jnp.where(cond, dsh - t, disp)
            return v

        def center_step(i, _):
            cxi = cxs[pl.ds(i, 16)][0]
            cyi = cys[pl.ds(i, 16)][0]
            czi = czs[pl.ds(i, 16)][0]

            def chunk_step(c, os_):
                xv = xs[pl.ds(c * 16, 16)]
                yv = ys[pl.ds(c * 16, 16)]
                zv = zs[pl.ds(c * 16, 16)]
                dx = xv - cxi
                dy = yv - cyi
                dz = zv - czi
                d = dx * dx + dy * dy + dz * dz
                idxv = c * 16 + iota
                ms = [d <= r2[j] for j in range(nbr)]
                rs = [jnp.where(m, 1, 0) for m in ms]
                for t in (1, 2, 4, 8):
                    for j in range(nbr):
                        shs[pl.ds(j * 48 + 16, 16)] = rs[j]
                    for j in range(nbr):
                        rs[j] = rs[j] + shs[pl.ds(j * 48 + 16 - t, 16)]
                new_os = []
                for j in range(nbr):
                    k = ks[j]
                    oj = os_[j]
                    cnt = rs[j][15]

                    def wr(o, m=ms[j], j=j, k=k, idxv=idxv, r=rs[j],
                           cnt=cnt):
                        disp = jnp.where(m, iota - (r - 1), 0)
                        v = compact16(idxv, disp)
                        bufs[j][pl.ds(i * (k + 16) + o, 16)] = v
                        return jnp.minimum(o + cnt, k)

                    new_os.append(
                        lax.cond((cnt > 0) & (oj < k), wr, lambda o: o, oj))
                return tuple(new_os)

            os_ = lax.fori_loop(0, nch, chunk_step,
                                tuple(jnp.int32(0) for _ in range(nbr)))
            for j in range(nbr):
                k = ks[j]
                first = bufs[j][pl.ds(i * (k + 16), 16)][0]
                for kc in range(k // 16):
                    pos = kc * 16 + iota
                    cur = bufs[j][pl.ds(i * (k + 16) + kc * 16, 16)]
                    cbufs[j][pl.ds(i * k + kc * 16, 16)] = jnp.where(
                        pos >= os_[j], first, cur)
            return 0

        lax.fori_loop(0, per, center_step, 0)
        for j in range(nbr):
            k = ks[j]
            pltpu.sync_copy(cbufs[j],
                            out_hs[j].at[pl.ds((b * s + lo) * k, per * k)])

    fn = pl.kernel(
        body,
        out_type=[jax.ShapeDtypeStruct((B * s * k,), jnp.int32) for k in ks],
        mesh=mesh,
        scratch_types=(
            [pltpu.VMEM((n,), jnp.float32)] * 3
            + [pltpu.VMEM((per + 16,), jnp.float32)] * 3
            + [pltpu.VMEM((per * (k + 16),), jnp.int32) for k in ks]
            + [pltpu.VMEM((per * k,), jnp.int32) for k in ks]
            + [pltpu.VMEM((144,), jnp.int32)]),
    )
    outs = fn(px, py, pz, cx, cy, cz)
    return [o.reshape(B, s, k) for o, k in zip(outs, ks)]


def _gather_sc(table, idx):
    bb, n, d = table.shape
    m = idx.shape[1]
    nw = 32
    rpw = (bb * m) // nw
    nchunk = rpw // 128
    mesh = plsc.VectorSubcoreMesh(core_axis_name="c", subcore_axis_name="s")
    idx_flat = idx.reshape(bb * m)

    def body(tab_h, idx_h, out_h, idx_v, rows_v, sem):
        w = lax.axis_index("s") * 2 + lax.axis_index("c")
        base = w * rpw
        b = base // m

        def chunk(c, _):
            off = base + c * 128
            pltpu.sync_copy(idx_h.at[pl.ds(off, 128)], idx_v)
            pltpu.async_copy(tab_h.at[b].at[idx_v], rows_v, sem).wait()
            pltpu.sync_copy(rows_v, out_h.at[pl.ds(off, 128)])
            return 0

        lax.fori_loop(0, nchunk, chunk, 0)

    fn = pl.kernel(
        body,
        out_type=jax.ShapeDtypeStruct((bb * m, d), jnp.float32),
        mesh=mesh,
        scratch_types=[
            pltpu.VMEM((128,), jnp.int32),
            pltpu.VMEM((128, d), jnp.float32),
            pltpu.SemaphoreType.DMA,
        ],
    )
    return fn(table, idx_flat).reshape(bb, m, d)


def _dense_body(x_ref, w_ref, b_ref, o_ref):
    o_ref[0] = (jnp.dot(x_ref[0], w_ref[...],
                        preferred_element_type=jnp.float32) + b_ref[...])


def _dense(x, w, b):
    s, cin = x.shape[1], x.shape[2]
    cout = w.shape[1]
    return pl.pallas_call(
        _dense_body,
        grid=(B,),
        in_specs=[
            pl.BlockSpec((1, s, cin), lambda bi: (bi, 0, 0)),
            pl.BlockSpec(w.shape, lambda bi: (0, 0)),
            pl.BlockSpec(b.shape, lambda bi: (0, 0)),
        ],
        out_specs=pl.BlockSpec((1, s, cout), lambda bi: (bi, 0, 0)),
        out_shape=jax.ShapeDtypeStruct((B, s, cout), jnp.float32),
    )(x, w, b)


def _ball_glue(r, k, xyz, new_xyz):
    n = xyz.shape[1]
    sqr = jnp.sum((new_xyz[:, :, None, :] - xyz[:, None, :, :]) ** 2, axis=-1)
    gid = jnp.where(sqr > r * r, n,
                    jnp.arange(n, dtype=jnp.int32)[None, None, :])
    gid = jnp.sort(gid, axis=-1)[:, :, :k]
    first = gid[:, :, 0:1]
    return jnp.where(gid == n, first, gid)


def _gather_pts(pts, idx):
    bidx = jnp.arange(pts.shape[0]).reshape((-1,) + (1,) * (idx.ndim - 1))
    return pts[bidx, idx]


def kernel(points, features, class_ids, params):
    x, y, z = points[:, 0], points[:, 1], points[:, 2]
    f0t = jnp.transpose(features, (0, 2, 1))
    xyz = jnp.stack([x, y, z], axis=-1)
    oh = jax.nn.one_hot(class_ids, NUM_CLASS_K, dtype=jnp.float32)[:, None, :]

    ms1 = [[_fold_cbr(p) for p in mlp] for mlp in params['ms1']]
    ms2 = [[_fold_cbr(p) for p in mlp] for mlp in params['ms2']]
    gsa_p = [_fold_cbr(p) for p in params['gsa']]
    fp1_p = [_fold_cbr(p) for p in params['fp1']]
    fp2_p = [_fold_cbr(p) for p in params['fp2']]
    fp3_p = [_fold_cbr(p) for p in params['fp3']]
    wc1, bc1 = _fold_cbr(params['cls1'])
    wc2 = params['cls2']['W']
    bc2 = params['cls2']['b'][None, :]

    _, nx, ny, nz = _fps(x, y, z, 512)
    l1_xyz = jnp.stack([nx, ny, nz], axis=-1)
    idxs1 = _ballq_sc(x, y, z, nx, ny, nz, [0.1, 0.2, 0.4], [32, 64, 128])
    t8 = jnp.concatenate([f0t, xyz, jnp.zeros((B, N0, 2), jnp.float32)],
                         axis=-1)
    outs = []
    for idx, k, mlp in zip(idxs1, [32, 64, 128], ms1):
        w1 = mlp[0][0]
        h1 = w1.shape[1]
        w1p = _padrc(w1, 8, 128)
        p1 = _dense(t8, w1p, _padc(mlp[0][1], 128))
        g = _gather_sc(p1, idx.reshape(B, 512 * k))
        w2p = _padrc(mlp[1][0], 128, mlp[1][0].shape[1])
        outs.append(_mlp_max(g, l1_xyz, _padc(w1[3:6], 128),
                             [w2p, mlp[2][0]], [mlp[1][1], mlp[2][1]],
                             512, k))
    l1_f = jnp.concatenate(outs, axis=-1)

    _, nx2, ny2, nz2 = _fps(nx, ny, nz, 128)
    l2_xyz = jnp.stack([nx2, ny2, nz2], axis=-1)
    l1_cat = jnp.concatenate(
        [l1_f, l1_xyz, jnp.zeros((B, 512, 61), jnp.float32)],
        axis=-1)
    idxs2 = _ballq_sc(nx, ny, nz, nx2, ny2, nz2, [0.4, 0.8], [64, 128])
    outs2 = []
    for idx, k, mlp in zip(idxs2, [64, 128], ms2):
        w1 = mlp[0][0]
        w1p = jnp.concatenate([w1, jnp.zeros((61, w1.shape[1]), jnp.float32)],
                              axis=0)
        proj1 = _dense(l1_cat, w1p, mlp[0][1])
        gp = _gather_sc(proj1, idx.reshape(B, 128 * k))
        outs2.append(_mlp_max(gp, l2_xyz, w1[320:323],
                              [w for w, _ in mlp[1:]],
                              [b for _, b in mlp[1:]], 128, k))
    l2_f = jnp.concatenate(outs2, axis=-1)

    x2 = jnp.concatenate([l2_f, l2_xyz], axis=-1)
    gvec = _gsa(x2, [w for w, _ in gsa_p], [b for _, b in gsa_p])

    l2_xyzt = jnp.stack([nx2, ny2, nz2], axis=1)
    l1_xyzt = jnp.stack([nx, ny, nz], axis=1)
    l2_fn = _fp1(l2_f, gvec, fp1_p[0][0], fp1_p[0][1],
                 fp1_p[1][0], fp1_p[1][1])
    l1_fn = _fp2(l1_xyz, l2_xyzt, l1_f, l2_fn,
                 fp2_p[0][0], fp2_p[0][1], fp2_p[1][0], fp2_p[1][1])
    out = _fp3_head(xyz, l1_xyzt, oh, f0t, l1_fn,
                    fp3_p[0][0], fp3_p[0][1], fp3_p[1][0], fp3_p[1][1],
                    wc1, bc1, wc2, bc2)
    return out
